# Initial kernel scaffold; baseline (speedup 1.0000x reference)
import jax
import jax.numpy as jnp
from jax.experimental import pallas as pl


def kernel(input_ids, word_emb, pos_emb, type_emb, ln_gamma, ln_beta):
    raise NotImplementedError("write your pallas kernel here")


# speedup vs baseline: 1.9905x; 1.9905x over previous
import functools

import jax
import jax.numpy as jnp
from jax import lax
from jax.experimental import pallas as pl
from jax.experimental.pallas import tpu as pltpu
from jax.experimental.pallas import tpu_sc as plsc

VOCAB = 100000
HID = 128
L = 200
B = 1024
EPS = 1e-12

NW = 32
TOK = B * L
PER_W = TOK // NW
CHUNK = 128
NCHUNK = PER_W // CHUNK
NH = HID // 16


def _tree_sum(vs):
    while len(vs) > 1:
        vs = [vs[i] + vs[i + 1] for i in range(0, len(vs) - 1, 2)] + (
            [vs[-1]] if len(vs) % 2 else [])
    return vs[0]


_GATHER_DNUMS = lax.GatherDimensionNumbers(
    offset_dims=(), collapsed_slice_dims=(0,), start_index_map=(0,))


def _shuffle(v, idx):
    return lax.gather(v, idx[:, None], _GATHER_DNUMS, slice_sizes=(1,),
                      mode=lax.GatherScatterMode.PROMISE_IN_BOUNDS)


def _xlane_sum(v):
    lanes = lax.iota(jnp.int32, 16)
    for sh in (8, 4, 2, 1):
        v = v + _shuffle(v, lanes ^ sh)
    return v


def _rsqrt(x):
    i = lax.bitcast_convert_type(x, jnp.int32)
    i = jnp.int32(0x5F3759DF) - (i >> 1)
    y = lax.bitcast_convert_type(i, jnp.float32)
    for _ in range(3):
        y = y * (1.5 - 0.5 * x * y * y)
    return y


def _sc_body(ids_hbm, word_hbm, pos_hbm, type_hbm, gamma_hbm, beta_hbm,
             out_hbm, idx_v, comb_v, type_v, gb_v, rows_v, sem):
    c = lax.axis_index("c")
    s = lax.axis_index("s")
    wid = s * 2 + c

    pltpu.sync_copy(ids_hbm.at[pl.ds(wid * PER_W, PER_W)], idx_v)
    pltpu.sync_copy(pos_hbm.at[pl.ds(0, L)], comb_v)
    pltpu.sync_copy(type_hbm, type_v)
    pltpu.sync_copy(gamma_hbm, gb_v.at[0])
    pltpu.sync_copy(beta_hbm, gb_v.at[1])

    def add_type(l, carry):
        for h in range(NH):
            sl = pl.ds(h * 16, 16)
            comb_v[l, sl] = comb_v[l, sl] + type_v[0, sl]
        return carry

    lax.fori_loop(0, L, add_type, 0)

    inv_h = jnp.float32(1.0 / HID)

    def chunk(g, carry):
        pltpu.async_copy(
            word_hbm.at[idx_v.at[pl.ds(g * CHUNK, CHUNK)]], rows_v, sem
        ).wait()

        base = wid * PER_W + g * CHUNK
        p0 = lax.rem(base, L)

        def row(i, p):
            x = [rows_v[i, pl.ds(h * 16, 16)] + comb_v[p, pl.ds(h * 16, 16)]
                 for h in range(NH)]
            mean = _xlane_sum(_tree_sum(x)) * inv_h
            t = [xh - mean for xh in x]
            var = _xlane_sum(_tree_sum([th * th for th in t])) * inv_h
            inv = _rsqrt(var + EPS)
            for h in range(NH):
                sl = pl.ds(h * 16, 16)
                rows_v[i, sl] = t[h] * inv * gb_v[0, sl] + gb_v[1, sl]
            p1 = p + 1
            return jnp.where(p1 >= L, 0, p1)

        lax.fori_loop(0, CHUNK, row, p0)
        pltpu.sync_copy(rows_v, out_hbm.at[pl.ds(base, CHUNK)])
        return carry

    lax.fori_loop(0, NCHUNK, chunk, 0)


@jax.jit
def _run(ids2, word_emb, pos_l, type_emb, ln_gamma, ln_beta):
    mesh = plsc.VectorSubcoreMesh(core_axis_name="c", subcore_axis_name="s")
    k = functools.partial(
        pl.kernel,
        mesh=mesh,
        out_type=jax.ShapeDtypeStruct((TOK, HID), jnp.float32),
        scratch_types=[
            pltpu.VMEM((PER_W,), jnp.int32),
            pltpu.VMEM((L, HID), jnp.float32),
            pltpu.VMEM((2, HID), jnp.float32),
            pltpu.VMEM((2, HID), jnp.float32),
            pltpu.VMEM((CHUNK, HID), jnp.float32),
            pltpu.SemaphoreType.DMA,
        ],
    )(_sc_body)
    return k(ids2, word_emb, pos_l, type_emb, ln_gamma, ln_beta)


def kernel(input_ids, word_emb, pos_emb, type_emb, ln_gamma, ln_beta):
    ids2 = input_ids.reshape(TOK).astype(jnp.int32)
    out = _run(ids2, word_emb, pos_emb[:L], type_emb, ln_gamma, ln_beta)
    return out.reshape(B, L, HID)


# speedup vs baseline: 4.0519x; 2.0356x over previous
import functools

import jax
import jax.numpy as jnp
from jax import lax
from jax.experimental import pallas as pl
from jax.experimental.pallas import tpu as pltpu
from jax.experimental.pallas import tpu_sc as plsc

VOCAB = 100000
HID = 128
L = 200
B = 1024
EPS = 1e-12

NW = 32
TOK = B * L
PER_W = TOK // NW
CHUNK = 128
NCHUNK = PER_W // CHUNK
NH = HID // 16


def _tree_sum(vs):
    while len(vs) > 1:
        vs = [vs[i] + vs[i + 1] for i in range(0, len(vs) - 1, 2)] + (
            [vs[-1]] if len(vs) % 2 else [])
    return vs[0]


_GATHER_DNUMS = lax.GatherDimensionNumbers(
    offset_dims=(), collapsed_slice_dims=(0,), start_index_map=(0,))


def _shuffle(v, idx):
    return lax.gather(v, idx[:, None], _GATHER_DNUMS, slice_sizes=(1,),
                      mode=lax.GatherScatterMode.PROMISE_IN_BOUNDS)


def _xlane_sum(v):
    lanes = lax.iota(jnp.int32, 16)
    for sh in (8, 4, 2, 1):
        v = v + _shuffle(v, lanes ^ sh)
    return v


def _rsqrt(x):
    i = lax.bitcast_convert_type(x, jnp.int32)
    i = jnp.int32(0x5F3759DF) - (i >> 1)
    y = lax.bitcast_convert_type(i, jnp.float32)
    for _ in range(3):
        y = y * (1.5 - 0.5 * x * y * y)
    return y


def _sc_body(ids_hbm, word_hbm, pos_hbm, type_hbm, gamma_hbm, beta_hbm,
             out_hbm, idx_v, comb_v, type_v, gb_v, rows_v, sem):
    c = lax.axis_index("c")
    s = lax.axis_index("s")
    wid = s * 2 + c

    pltpu.sync_copy(ids_hbm.at[pl.ds(wid * PER_W, PER_W)], idx_v)
    pltpu.sync_copy(pos_hbm.at[pl.ds(0, L)], comb_v)
    pltpu.sync_copy(type_hbm, type_v)
    pltpu.sync_copy(gamma_hbm, gb_v.at[0])
    pltpu.sync_copy(beta_hbm, gb_v.at[1])

    def add_type(l, carry):
        for h in range(NH):
            sl = pl.ds(h * 16, 16)
            comb_v[l, sl] = comb_v[l, sl] + type_v[0, sl]
        return carry

    lax.fori_loop(0, L, add_type, 0)

    inv_h = jnp.float32(1.0 / HID)

    def chunk(g, carry):
        pltpu.async_copy(
            word_hbm.at[idx_v.at[pl.ds(g * CHUNK, CHUNK)]], rows_v, sem
        ).wait()

        base = wid * PER_W + g * CHUNK
        p0 = lax.rem(base, L)

        @plsc.parallel_loop(0, CHUNK, unroll=2)
        def row(i):
            p = lax.rem(p0 + i, L)
            x = [rows_v[i, pl.ds(h * 16, 16)] + comb_v[p, pl.ds(h * 16, 16)]
                 for h in range(NH)]
            ssum = _xlane_sum(_tree_sum(x)) * inv_h
            qsum = _xlane_sum(_tree_sum([xh * xh for xh in x])) * inv_h
            inv = _rsqrt(qsum - ssum * ssum + EPS)
            for h in range(NH):
                sl = pl.ds(h * 16, 16)
                rows_v[i, sl] = (x[h] - ssum) * inv * gb_v[0, sl] + gb_v[1, sl]
        pltpu.sync_copy(rows_v, out_hbm.at[pl.ds(base, CHUNK)])
        return carry

    lax.fori_loop(0, NCHUNK, chunk, 0)


@jax.jit
def _run(ids2, word_emb, pos_l, type_emb, ln_gamma, ln_beta):
    mesh = plsc.VectorSubcoreMesh(core_axis_name="c", subcore_axis_name="s")
    k = functools.partial(
        pl.kernel,
        mesh=mesh,
        out_type=jax.ShapeDtypeStruct((TOK, HID), jnp.float32),
        scratch_types=[
            pltpu.VMEM((PER_W,), jnp.int32),
            pltpu.VMEM((L, HID), jnp.float32),
            pltpu.VMEM((2, HID), jnp.float32),
            pltpu.VMEM((2, HID), jnp.float32),
            pltpu.VMEM((CHUNK, HID), jnp.float32),
            pltpu.SemaphoreType.DMA,
        ],
    )(_sc_body)
    return k(ids2, word_emb, pos_l, type_emb, ln_gamma, ln_beta)


def kernel(input_ids, word_emb, pos_emb, type_emb, ln_gamma, ln_beta):
    ids2 = input_ids.reshape(TOK).astype(jnp.int32)
    out = _run(ids2, word_emb, pos_emb[:L], type_emb, ln_gamma, ln_beta)
    return out.reshape(B, L, HID)


# speedup vs baseline: 5.9438x; 1.4669x over previous
import functools

import jax
import jax.numpy as jnp
from jax import lax
from jax.experimental import pallas as pl
from jax.experimental.pallas import tpu as pltpu
from jax.experimental.pallas import tpu_sc as plsc

VOCAB = 100000
HID = 128
L = 200
B = 1024
EPS = 1e-12

NW = 32
TOK = B * L
PER_W = TOK // NW
CHUNK = 128
NCHUNK = PER_W // CHUNK
NH = HID // 16


def _tree_sum(vs):
    while len(vs) > 1:
        vs = [vs[i] + vs[i + 1] for i in range(0, len(vs) - 1, 2)] + (
            [vs[-1]] if len(vs) % 2 else [])
    return vs[0]


_GATHER_DNUMS = lax.GatherDimensionNumbers(
    offset_dims=(), collapsed_slice_dims=(0,), start_index_map=(0,))


def _shuffle(v, idx):
    return lax.gather(v, idx[:, None], _GATHER_DNUMS, slice_sizes=(1,),
                      mode=lax.GatherScatterMode.PROMISE_IN_BOUNDS)


def _xlane_sum(v):
    lanes = lax.iota(jnp.int32, 16)
    for sh in (8, 4, 2, 1):
        v = v + _shuffle(v, lanes ^ sh)
    return v


def _rsqrt(x):
    i = lax.bitcast_convert_type(x, jnp.int32)
    i = jnp.int32(0x5F3759DF) - (i >> 1)
    y = lax.bitcast_convert_type(i, jnp.float32)
    for _ in range(2):
        y = y * (1.5 - 0.5 * x * y * y)
    return y


def _sc_body(ids_hbm, word_hbm, pos_hbm, type_hbm, gamma_hbm, beta_hbm,
             out_hbm, idx_v, comb_v, type_v, gb_v, rows_v, sem):
    c = lax.axis_index("c")
    s = lax.axis_index("s")
    wid = s * 2 + c

    pltpu.sync_copy(ids_hbm.at[pl.ds(wid * PER_W, PER_W)], idx_v)
    pltpu.sync_copy(pos_hbm.at[pl.ds(0, L)], comb_v)
    pltpu.sync_copy(type_hbm, type_v)
    pltpu.sync_copy(gamma_hbm, gb_v.at[0])
    pltpu.sync_copy(beta_hbm, gb_v.at[1])

    def add_type(l, carry):
        for h in range(NH):
            sl = pl.ds(h * 16, 16)
            comb_v[l, sl] = comb_v[l, sl] + type_v[0, sl]
        return carry

    lax.fori_loop(0, L, add_type, 0)

    inv_h = jnp.float32(1.0 / HID)

    def chunk(g, carry):
        pltpu.async_copy(
            word_hbm.at[idx_v.at[pl.ds(g * CHUNK, CHUNK)]], rows_v, sem
        ).wait()

        base = wid * PER_W + g * CHUNK
        p0 = lax.rem(base, L)

        @plsc.parallel_loop(0, CHUNK, unroll=1)
        def row(i):
            p = lax.rem(p0 + i, L)
            x = [rows_v[i, pl.ds(h * 16, 16)] + comb_v[p, pl.ds(h * 16, 16)]
                 for h in range(NH)]
            ssum = _xlane_sum(_tree_sum(x)) * inv_h
            qsum = _xlane_sum(_tree_sum([xh * xh for xh in x])) * inv_h
            inv = _rsqrt(qsum - ssum * ssum + EPS)
            for h in range(NH):
                sl = pl.ds(h * 16, 16)
                rows_v[i, sl] = (x[h] - ssum) * inv * gb_v[0, sl] + gb_v[1, sl]
        pltpu.sync_copy(rows_v, out_hbm.at[pl.ds(base, CHUNK)])
        return carry

    lax.fori_loop(0, NCHUNK, chunk, 0)


@jax.jit
def _run(ids2, word_emb, pos_l, type_emb, ln_gamma, ln_beta):
    mesh = plsc.VectorSubcoreMesh(core_axis_name="c", subcore_axis_name="s")
    k = functools.partial(
        pl.kernel,
        mesh=mesh,
        out_type=jax.ShapeDtypeStruct((TOK, HID), jnp.float32),
        scratch_types=[
            pltpu.VMEM((PER_W,), jnp.int32),
            pltpu.VMEM((L, HID), jnp.float32),
            pltpu.VMEM((2, HID), jnp.float32),
            pltpu.VMEM((2, HID), jnp.float32),
            pltpu.VMEM((CHUNK, HID), jnp.float32),
            pltpu.SemaphoreType.DMA,
        ],
    )(_sc_body)
    return k(ids2, word_emb, pos_l, type_emb, ln_gamma, ln_beta)


def kernel(input_ids, word_emb, pos_emb, type_emb, ln_gamma, ln_beta):
    ids2 = input_ids.reshape(TOK).astype(jnp.int32)
    out = _run(ids2, word_emb, pos_emb[:L], type_emb, ln_gamma, ln_beta)
    return out.reshape(B, L, HID)


# speedup vs baseline: 9.2967x; 1.5641x over previous
import functools

import jax
import jax.numpy as jnp
from jax import lax
from jax.experimental import pallas as pl
from jax.experimental.pallas import tpu as pltpu
from jax.experimental.pallas import tpu_sc as plsc

VOCAB = 100000
HID = 128
L = 200
B = 1024
EPS = 1e-12

NW = 32
TOK = B * L
PER_W = TOK // NW
CHUNK = 128
NCHUNK = PER_W // CHUNK
NBUF = 5
NH = HID // 16


def _tree_sum(vs):
    while len(vs) > 1:
        vs = [vs[i] + vs[i + 1] for i in range(0, len(vs) - 1, 2)] + (
            [vs[-1]] if len(vs) % 2 else [])
    return vs[0]


_GATHER_DNUMS = lax.GatherDimensionNumbers(
    offset_dims=(), collapsed_slice_dims=(0,), start_index_map=(0,))


def _shuffle(v, idx):
    return lax.gather(v, idx[:, None], _GATHER_DNUMS, slice_sizes=(1,),
                      mode=lax.GatherScatterMode.PROMISE_IN_BOUNDS)


def _xlane_sum(v):
    lanes = lax.iota(jnp.int32, 16)
    for sh in (8, 4, 2, 1):
        v = v + _shuffle(v, lanes ^ sh)
    return v


def _rsqrt(x):
    i = lax.bitcast_convert_type(x, jnp.int32)
    i = jnp.int32(0x5F3759DF) - (i >> 1)
    y = lax.bitcast_convert_type(i, jnp.float32)
    for _ in range(2):
        y = y * (1.5 - 0.5 * x * y * y)
    return y


def _sc_body(ids_hbm, word_hbm, pos_hbm, type_hbm, gamma_hbm, beta_hbm,
             out_hbm, idx_v, comb_v, type_v, gb_v,
             rows0, rows1, rows2, rows3, rows4,
             sg0, sg1, sg2, sg3, sg4, so0, so1, so2, so3, so4):
    bufs = [rows0, rows1, rows2, rows3, rows4]
    sem_g = [sg0, sg1, sg2, sg3, sg4]
    sem_o = [so0, so1, so2, so3, so4]
    c = lax.axis_index("c")
    s = lax.axis_index("s")
    wid = s * 2 + c

    pltpu.sync_copy(ids_hbm.at[pl.ds(wid * PER_W, PER_W)], idx_v)
    pltpu.sync_copy(pos_hbm.at[pl.ds(0, L)], comb_v)
    pltpu.sync_copy(type_hbm, type_v)
    pltpu.sync_copy(gamma_hbm, gb_v.at[0])
    pltpu.sync_copy(beta_hbm, gb_v.at[1])

    def add_type(l, carry):
        for h in range(NH):
            sl = pl.ds(h * 16, 16)
            comb_v[l, sl] = comb_v[l, sl] + type_v[0, sl]
        return carry

    lax.fori_loop(0, L, add_type, 0)

    inv_h = jnp.float32(1.0 / HID)

    def issue_gather(g, b):
        pltpu.async_copy(
            word_hbm.at[idx_v.at[pl.ds(g * CHUNK, CHUNK)]], bufs[b], sem_g[b])

    def process(g, b):
        rows_v = bufs[b]
        base = wid * PER_W + g * CHUNK
        p0 = lax.rem(base, L)

        @plsc.parallel_loop(0, CHUNK, unroll=1)
        def row(i):
            p = lax.rem(p0 + i, L)
            x = [rows_v[i, pl.ds(h * 16, 16)] + comb_v[p, pl.ds(h * 16, 16)]
                 for h in range(NH)]
            ssum = _xlane_sum(_tree_sum(x)) * inv_h
            qsum = _xlane_sum(_tree_sum([xh * xh for xh in x])) * inv_h
            inv = _rsqrt(qsum - ssum * ssum + EPS)
            for h in range(NH):
                sl = pl.ds(h * 16, 16)
                rows_v[i, sl] = (x[h] - ssum) * inv * gb_v[0, sl] + gb_v[1, sl]

    issue_gather(0, 0)
    issue_gather(1, 1)

    def super_chunk(gq, carry):
        for j in range(NBUF):
            g = gq * NBUF + j
            pltpu.make_async_copy(
                word_hbm.at[idx_v.at[pl.ds(g * CHUNK, CHUNK)]],
                bufs[j], sem_g[j]).wait()
            process(g, j)
            base = wid * PER_W + g * CHUNK
            pltpu.async_copy(
                bufs[j], out_hbm.at[pl.ds(base, CHUNK)], sem_o[j])
            b2 = (j + 2) % NBUF

            @pl.when(g + 2 < NCHUNK)
            def _prefetch():
                @pl.when(g >= NBUF - 2)
                def _drain_wb():
                    pltpu.make_async_copy(
                        bufs[b2], out_hbm.at[pl.ds(0, CHUNK)],
                        sem_o[b2]).wait()
                issue_gather(g + 2, b2)
        return carry

    lax.fori_loop(0, NCHUNK // NBUF, super_chunk, 0)
    for b in range(NBUF):
        pltpu.make_async_copy(
            bufs[b], out_hbm.at[pl.ds(0, CHUNK)], sem_o[b]).wait()


@jax.jit
def _run(ids2, word_emb, pos_l, type_emb, ln_gamma, ln_beta):
    mesh = plsc.VectorSubcoreMesh(core_axis_name="c", subcore_axis_name="s")
    k = functools.partial(
        pl.kernel,
        mesh=mesh,
        out_type=jax.ShapeDtypeStruct((TOK, HID), jnp.float32),
        scratch_types=[
            pltpu.VMEM((PER_W,), jnp.int32),
            pltpu.VMEM((L, HID), jnp.float32),
            pltpu.VMEM((2, HID), jnp.float32),
            pltpu.VMEM((2, HID), jnp.float32),
        ] + [pltpu.VMEM((CHUNK, HID), jnp.float32) for _ in range(NBUF)]
          + [pltpu.SemaphoreType.DMA for _ in range(2 * NBUF)],
    )(_sc_body)
    return k(ids2, word_emb, pos_l, type_emb, ln_gamma, ln_beta)


def kernel(input_ids, word_emb, pos_emb, type_emb, ln_gamma, ln_beta):
    ids2 = input_ids.reshape(TOK).astype(jnp.int32)
    out = _run(ids2, word_emb, pos_emb[:L], type_emb, ln_gamma, ln_beta)
    return out.reshape(B, L, HID)
